# baseline (speedup 1.0000x reference)
import jax
import jax.numpy as jnp
from jax import lax
from jax.experimental import pallas as pl
from jax.experimental.pallas import tpu as pltpu
from jax.experimental.pallas import tpu_sc as plsc

N = 2048
E = 8192
D = 128
K = 16
NS = 16
RPS = N // NS
SLOTS = RPS * K
ROWBLK = 128
OUTPAD = 16


def _tc_body(adj_ref, atoms_ref, hic_ref, cnt_ref, v0_ref):
    a = adj_ref[...]
    m = a > 0.0
    colj = lax.broadcasted_iota(jnp.int32, (ROWBLK, N), 1)
    cmst = jnp.where(m, colj, N)
    hcols = []
    cnt = jnp.zeros((ROWBLK,), jnp.int32)
    for k in range(K):
        mn = jnp.min(cmst, axis=1)
        valid = mn < N
        hcols.append(jnp.where(valid, mn, 0))
        cnt = cnt + valid.astype(jnp.int32)
        if k < K - 1:
            cmst = jnp.where(cmst == mn[:, None], N, cmst)
    hic_ref[...] = jnp.stack(hcols, axis=1)
    cnt_ref[...] = cnt
    v0_ref[...] = jnp.sum(atoms_ref[...], axis=1)


def _tc_extract(atoms, adj):
    grid = (N // ROWBLK,)
    return pl.pallas_call(
        _tc_body,
        grid=grid,
        in_specs=[
            pl.BlockSpec((ROWBLK, N), lambda i: (i, 0)),
            pl.BlockSpec((ROWBLK, D), lambda i: (i, 0)),
        ],
        out_specs=[
            pl.BlockSpec((ROWBLK, K), lambda i: (i, 0)),
            pl.BlockSpec((ROWBLK,), lambda i: (i,)),
            pl.BlockSpec((ROWBLK,), lambda i: (i,)),
        ],
        out_shape=[
            jax.ShapeDtypeStruct((N, K), jnp.int32),
            jax.ShapeDtypeStruct((N,), jnp.int32),
            jax.ShapeDtypeStruct((N,), jnp.float32),
        ],
    )(adj, atoms)


def _sc_body(adjf, hicf, cnth, v0h, out,
             vals, ecur, hloc, idx2, idxo2, gidx2, vhi, cntloc, rbfull,
             v0full,
             vown, v0own, hebrow, hp, zbuf, stage, pos2, pbuf, sem,
             v_sh, hebp_sh, ep_sh, out_sh):
    s = lax.axis_index("s")
    iota = lax.iota(jnp.int32, 16)
    zf = jnp.zeros((16,), jnp.float32)
    lane0 = iota == 0

    def shuf(x, perm):
        return jnp.take_along_axis(x, perm, axis=0)

    def splat(x, lane):
        return shuf(x, jnp.broadcast_to(lane, (16,)).astype(jnp.int32))

    def vsum(x):
        for k in (1, 2, 4, 8):
            x = x + shuf(x, jnp.bitwise_xor(iota, k))
        return x

    def prefix_incl(x):
        zero = jnp.zeros((16,), x.dtype)
        for k in (1, 2, 4, 8):
            sh = shuf(x, jnp.maximum(iota - k, 0))
            x = x + jnp.where(iota >= k, sh, zero)
        return x

    pltpu.sync_copy(hicf.at[pl.ds(SLOTS * s, SLOTS)], hloc)
    pltpu.sync_copy(cnth, cntloc)
    pltpu.sync_copy(v0h, v0full)
    pltpu.sync_copy(v0h.at[pl.ds(RPS * s, RPS)], vown)
    pltpu.sync_copy(v0h.at[pl.ds(RPS * s, RPS)], v0own)
    pltpu.sync_copy(vown, v_sh.at[pl.ds(RPS * s, RPS)])

    def zb(k, c):
        zbuf[pl.ds(16 * k, 16)] = zf
        return c

    lax.fori_loop(0, N // 16, zb, 0)

    def ib(j, c):
        def ik(k, c2):
            h16 = hloc[pl.ds(128 * j + 16 * k, 16)]
            idx2[j, pl.ds(16 * k, 16)] = h16
            idxo2[j, pl.ds(16 * k, 16)] = h16 + N * s
            gidx2[j, pl.ds(16 * k, 16)] = h16 + N * (RPS * s + 8 * j + k)
            return c2

        lax.fori_loop(0, 8, ik, 0)
        return c

    lax.fori_loop(0, NS, ib, 0)

    def vb(j, c):
        pltpu.async_copy(adjf.at[gidx2.at[j]],
                         vals.at[pl.ds(128 * j, 128)], sem)
        return c

    lax.fori_loop(0, NS, vb, 0)

    def vw(j, c):
        pltpu.make_async_copy(adjf.at[gidx2.at[j]],
                              vals.at[pl.ds(128 * j, 128)], sem).wait()
        return c

    lax.fori_loop(0, NS, vw, 0)

    def rv(k, a):
        return a + v0full[pl.ds(16 * k, 16)]

    u0 = vsum(lax.fori_loop(0, N // 16, rv, zf))
    u = u0
    t_run = u0
    plsc.subcore_barrier()

    for t in range(3):
        esrc = vals if t == 0 else ecur
        if t < 2:
            pltpu.sync_copy(zbuf, hebp_sh.at[pl.ds(N * s, N)])

        def gb(j, c):
            pltpu.async_copy(v_sh.at[idx2.at[j]],
                             vhi.at[pl.ds(128 * j, 128)], sem)
            return c

        lax.fori_loop(0, NS, gb, 0)

        def gw(j, c):
            pltpu.make_async_copy(v_sh.at[idx2.at[j]],
                                  vhi.at[pl.ds(128 * j, 128)], sem).wait()
            return c

        lax.fori_loop(0, NS, gw, 0)

        def eb(a, acc, esrc=esrc, u=u, t=t):
            vchunk = vown[pl.ds(16 * a, 16)]
            cchunk = cntloc[pl.ds(RPS * s + 16 * a, 16)]
            hr = zf
            for b in range(16):
                off = 256 * a + 16 * b
                vsp = splat(vchunk, b)
                eold = esrc[pl.ds(off, 16)]
                e0v = vals[pl.ds(off, 16)]
                vh = vhi[pl.ds(off, 16)]
                mm = jnp.where(iota < splat(cchunk, b), 1.0, 0.0)
                en = mm * (0.5 * eold + 0.5 * e0v + 0.01 * (vsp + vh)
                           + 0.001 * u)
                ecur[pl.ds(off, 16)] = en
                acc = acc + en
                if t < 2:
                    hr = jnp.where(iota == b, vsum(en), hr)
            if t < 2:
                hebrow[pl.ds(16 * a, 16)] = hr
            return acc

        acc = lax.fori_loop(0, RPS // 16, eb, zf)
        if t == 2:
            break

        def sb(j, c):
            pltpu.async_copy(ecur.at[pl.ds(128 * j, 128)],
                             hebp_sh.at[idxo2.at[j]], sem, add=True)
            return c

        lax.fori_loop(0, NS, sb, 0)

        def sw(j, c):
            pltpu.make_async_copy(ecur.at[pl.ds(128 * j, 128)],
                                  hebp_sh.at[idxo2.at[j]], sem).wait()
            return c

        lax.fori_loop(0, NS, sw, 0)
        stage[pl.ds(0, 16)] = jnp.where(lane0, vsum(acc), 0.0)
        pltpu.sync_copy(stage, ep_sh.at[pl.ds(16 * s, 16)])
        plsc.subcore_barrier()

        for r in range(NS):
            pltpu.async_copy(hebp_sh.at[pl.ds(N * r + RPS * s, RPS)],
                             hp.at[pl.ds(RPS * r, RPS)], sem)
        for r in range(NS):
            pltpu.make_async_copy(hebp_sh.at[pl.ds(N * r + RPS * s, RPS)],
                                  hp.at[pl.ds(RPS * r, RPS)], sem).wait()

        def nb(k, accv, u=u):
            hv = hebrow[pl.ds(16 * k, 16)]
            for r in range(NS):
                hv = hv + hp[pl.ds(RPS * r + 16 * k, 16)]
            vn = (0.5 * vown[pl.ds(16 * k, 16)]
                  + 0.5 * v0own[pl.ds(16 * k, 16)]
                  + (0.1 * D) * hv + 0.001 * u)
            vown[pl.ds(16 * k, 16)] = vn
            return accv + vn

        lax.fori_loop(0, RPS // 16, nb, zf)
        pltpu.sync_copy(vown, v_sh.at[pl.ds(RPS * s, RPS)])
        plsc.subcore_barrier()

        pltpu.sync_copy(ep_sh, pbuf)
        acc2 = zf
        for r in range(NS):
            acc2 = acc2 + pbuf[pl.ds(16 * r, 16)]
        heb_tot = vsum(acc2)
        t_tot = (0.5 * t_run + 0.5 * u0 + (0.2 * D) * heb_tot
                 + 0.001 * N * u)
        u = 0.5 * u + 0.5 * u0 + (0.01 * D) * heb_tot + 0.001 * t_tot
        t_run = t_tot

    def cs(k, carry):
        x = cntloc[pl.ds(16 * k, 16)]
        inc = prefix_incl(x)
        rbfull[pl.ds(16 * k, 16)] = (carry + inc) - x
        return carry + splat(inc, 15)

    lax.fori_loop(0, N // 16, cs, jnp.zeros((16,), jnp.int32))

    def pb(a, c):
        rbchunk = rbfull[pl.ds(RPS * s + 16 * a, 16)]
        cchunk = cntloc[pl.ds(RPS * s + 16 * a, 16)]
        for b in range(16):
            off = 256 * a + 16 * b
            pos = jnp.where(iota < splat(cchunk, b),
                            splat(rbchunk, b) + iota,
                            jnp.broadcast_to(E, (16,)).astype(jnp.int32))
            j = 2 * a + (b // 8)
            pos2[j, pl.ds(16 * (b % 8), 16)] = pos
        return c

    lax.fori_loop(0, RPS // 16, pb, 0)

    def ob(j, c):
        pltpu.async_copy(ecur.at[pl.ds(128 * j, 128)],
                         out_sh.at[pos2.at[j]], sem)
        return c

    lax.fori_loop(0, NS, ob, 0)

    def ow(j, c):
        pltpu.make_async_copy(ecur.at[pl.ds(128 * j, 128)],
                              out_sh.at[pos2.at[j]], sem).wait()
        return c

    lax.fori_loop(0, NS, ow, 0)
    plsc.subcore_barrier()
    pltpu.sync_copy(out_sh.at[pl.ds((E // NS) * s, E // NS)],
                    out.at[pl.ds((E // NS) * s, E // NS)])


def _sc_recur(adjf, hicf, cnt, v0):
    mesh = plsc.VectorSubcoreMesh(core_axis_name="c", subcore_axis_name="s",
                                  num_cores=1)
    fn = pl.kernel(
        _sc_body,
        out_type=jax.ShapeDtypeStruct((E + OUTPAD,), jnp.float32),
        mesh=mesh,
        scratch_types=[
            pltpu.VMEM((SLOTS,), jnp.float32),
            pltpu.VMEM((SLOTS,), jnp.float32),
            pltpu.VMEM((SLOTS,), jnp.int32),
            pltpu.VMEM((NS, 128), jnp.int32),
            pltpu.VMEM((NS, 128), jnp.int32),
            pltpu.VMEM((NS, 128), jnp.int32),
            pltpu.VMEM((SLOTS,), jnp.float32),
            pltpu.VMEM((N,), jnp.int32),
            pltpu.VMEM((N,), jnp.int32),
            pltpu.VMEM((N,), jnp.float32),
            pltpu.VMEM((RPS,), jnp.float32),
            pltpu.VMEM((RPS,), jnp.float32),
            pltpu.VMEM((RPS,), jnp.float32),
            pltpu.VMEM((N,), jnp.float32),
            pltpu.VMEM((N,), jnp.float32),
            pltpu.VMEM((16,), jnp.float32),
            pltpu.VMEM((NS, 128), jnp.int32),
            pltpu.VMEM((NS * 16,), jnp.float32),
            pltpu.SemaphoreType.DMA,
            pltpu.VMEM_SHARED((N,), jnp.float32),
            pltpu.VMEM_SHARED((NS * N,), jnp.float32),
            pltpu.VMEM_SHARED((NS * 16,), jnp.float32),
            pltpu.VMEM_SHARED((E + OUTPAD,), jnp.float32),
        ],
    )
    return fn(adjf, hicf, cnt, v0)


def kernel(atoms, adjacency_map):
    hic, cnt, v0 = _tc_extract(atoms, adjacency_map)
    out = _sc_recur(adjacency_map.reshape(-1), hic.reshape(-1), cnt, v0)
    return out[:E].reshape(E, 1)


# speedup vs baseline: 1.0097x; 1.0097x over previous
import jax
import jax.numpy as jnp
from jax import lax
from jax.experimental import pallas as pl
from jax.experimental.pallas import tpu as pltpu
from jax.experimental.pallas import tpu_sc as plsc

N = 2048
E = 8192
D = 128
K = 16
NS = 16
RPS = N // NS
SLOTS = RPS * K
ROWBLK = 128
OUTPAD = 16


def _tc_body(adj_ref, atoms_ref, hic_ref, cnt_ref, v0_ref):
    a = adj_ref[...]
    m = a > 0.0
    colj = lax.broadcasted_iota(jnp.int32, (ROWBLK, N), 1)
    cmst = jnp.where(m, colj, N)
    hcols = []
    cnt = jnp.zeros((ROWBLK,), jnp.int32)
    for k in range(K):
        mn = jnp.min(cmst, axis=1)
        valid = mn < N
        hcols.append(jnp.where(valid, mn, 0))
        cnt = cnt + valid.astype(jnp.int32)
        if k < K - 1:
            cmst = jnp.where(cmst == mn[:, None], N, cmst)
    hic_ref[...] = jnp.stack(hcols, axis=1)
    cnt_ref[...] = cnt
    v0_ref[...] = jnp.sum(atoms_ref[...], axis=1)


def _tc_extract(atoms, adj):
    grid = (N // ROWBLK,)
    return pl.pallas_call(
        _tc_body,
        grid=grid,
        in_specs=[
            pl.BlockSpec((ROWBLK, N), lambda i: (i, 0)),
            pl.BlockSpec((ROWBLK, D), lambda i: (i, 0)),
        ],
        out_specs=[
            pl.BlockSpec((ROWBLK, K), lambda i: (i, 0)),
            pl.BlockSpec((ROWBLK,), lambda i: (i,)),
            pl.BlockSpec((ROWBLK,), lambda i: (i,)),
        ],
        out_shape=[
            jax.ShapeDtypeStruct((N, K), jnp.int32),
            jax.ShapeDtypeStruct((N,), jnp.int32),
            jax.ShapeDtypeStruct((N,), jnp.float32),
        ],
    )(adj, atoms)


def _sc_body(adjf, hicf, cnth, v0h, out,
             vals, ecur, hloc, idx2, idxo2, gidx2, vhi, cntloc, rbfull,
             v0full,
             vown, v0own, hebrow, hp, zbuf, stage, pos2, pbuf, sem,
             v_sh, hebp_sh, ep_sh, out_sh):
    s = lax.axis_index("s")
    iota = lax.iota(jnp.int32, 16)
    zf = jnp.zeros((16,), jnp.float32)
    lane0 = iota == 0

    def shuf(x, perm):
        return jnp.take_along_axis(x, perm, axis=0)

    def splat(x, lane):
        return shuf(x, jnp.broadcast_to(lane, (16,)).astype(jnp.int32))

    def vsum(x):
        for k in (1, 2, 4, 8):
            x = x + shuf(x, jnp.bitwise_xor(iota, k))
        return x

    def prefix_incl(x):
        zero = jnp.zeros((16,), x.dtype)
        for k in (1, 2, 4, 8):
            sh = shuf(x, jnp.maximum(iota - k, 0))
            x = x + jnp.where(iota >= k, sh, zero)
        return x

    pltpu.async_copy(hicf.at[pl.ds(SLOTS * s, SLOTS)], hloc, sem)
    pltpu.async_copy(cnth, cntloc, sem)
    pltpu.async_copy(v0h, v0full, sem)
    pltpu.async_copy(v0h.at[pl.ds(RPS * s, RPS)], vown, sem)
    pltpu.async_copy(v0h.at[pl.ds(RPS * s, RPS)], v0own, sem)
    pltpu.make_async_copy(hicf.at[pl.ds(SLOTS * s, SLOTS)], hloc, sem).wait()
    pltpu.make_async_copy(cnth, cntloc, sem).wait()
    pltpu.make_async_copy(v0h, v0full, sem).wait()
    pltpu.make_async_copy(v0h.at[pl.ds(RPS * s, RPS)], vown, sem).wait()
    pltpu.make_async_copy(v0h.at[pl.ds(RPS * s, RPS)], v0own, sem).wait()
    pltpu.sync_copy(vown, v_sh.at[pl.ds(RPS * s, RPS)])

    def zb(k, c):
        zbuf[pl.ds(16 * k, 16)] = zf
        return c

    lax.fori_loop(0, N // 16, zb, 0)

    def ib(j, c):
        def ik(k, c2):
            h16 = hloc[pl.ds(128 * j + 16 * k, 16)]
            idx2[j, pl.ds(16 * k, 16)] = h16
            idxo2[j, pl.ds(16 * k, 16)] = h16 + N * s
            gidx2[j, pl.ds(16 * k, 16)] = h16 + N * (RPS * s + 8 * j + k)
            return c2

        lax.fori_loop(0, 8, ik, 0)
        return c

    lax.fori_loop(0, NS, ib, 0)

    def vb(j, c):
        pltpu.async_copy(adjf.at[gidx2.at[j]],
                         vals.at[pl.ds(128 * j, 128)], sem)
        return c

    lax.fori_loop(0, NS, vb, 0)

    def vw(j, c):
        pltpu.make_async_copy(adjf.at[gidx2.at[j]],
                              vals.at[pl.ds(128 * j, 128)], sem).wait()
        return c

    lax.fori_loop(0, NS, vw, 0)

    def rv(k, a):
        return a + v0full[pl.ds(16 * k, 16)]

    u0 = vsum(lax.fori_loop(0, N // 16, rv, zf))
    u = u0
    t_run = u0
    plsc.subcore_barrier()

    for t in range(3):
        esrc = vals if t == 0 else ecur
        if t < 2:
            pltpu.sync_copy(zbuf, hebp_sh.at[pl.ds(N * s, N)])

        def gb(j, c):
            pltpu.async_copy(v_sh.at[idx2.at[j]],
                             vhi.at[pl.ds(128 * j, 128)], sem)
            return c

        lax.fori_loop(0, NS, gb, 0)

        def gw(j, c):
            pltpu.make_async_copy(v_sh.at[idx2.at[j]],
                                  vhi.at[pl.ds(128 * j, 128)], sem).wait()
            return c

        lax.fori_loop(0, NS, gw, 0)

        def eb(a, acc, esrc=esrc, u=u, t=t):
            vchunk = vown[pl.ds(16 * a, 16)]
            cchunk = cntloc[pl.ds(RPS * s + 16 * a, 16)]
            hr = zf
            for b in range(16):
                off = 256 * a + 16 * b
                vsp = splat(vchunk, b)
                eold = esrc[pl.ds(off, 16)]
                e0v = vals[pl.ds(off, 16)]
                vh = vhi[pl.ds(off, 16)]
                mm = jnp.where(iota < splat(cchunk, b), 1.0, 0.0)
                en = mm * (0.5 * eold + 0.5 * e0v + 0.01 * (vsp + vh)
                           + 0.001 * u)
                ecur[pl.ds(off, 16)] = en
                acc = acc + en
                if t < 2:
                    hr = jnp.where(iota == b, vsum(en), hr)
            if t < 2:
                hebrow[pl.ds(16 * a, 16)] = hr
            return acc

        acc = lax.fori_loop(0, RPS // 16, eb, zf)
        if t == 2:
            break

        def sb(j, c):
            pltpu.async_copy(ecur.at[pl.ds(128 * j, 128)],
                             hebp_sh.at[idxo2.at[j]], sem, add=True)
            return c

        lax.fori_loop(0, NS, sb, 0)

        def sw(j, c):
            pltpu.make_async_copy(ecur.at[pl.ds(128 * j, 128)],
                                  hebp_sh.at[idxo2.at[j]], sem).wait()
            return c

        lax.fori_loop(0, NS, sw, 0)
        stage[pl.ds(0, 16)] = jnp.where(lane0, vsum(acc), 0.0)
        pltpu.sync_copy(stage, ep_sh.at[pl.ds(16 * s, 16)])
        plsc.subcore_barrier()

        for r in range(NS):
            pltpu.async_copy(hebp_sh.at[pl.ds(N * r + RPS * s, RPS)],
                             hp.at[pl.ds(RPS * r, RPS)], sem)
        for r in range(NS):
            pltpu.make_async_copy(hebp_sh.at[pl.ds(N * r + RPS * s, RPS)],
                                  hp.at[pl.ds(RPS * r, RPS)], sem).wait()

        def nb(k, accv, u=u):
            hv = hebrow[pl.ds(16 * k, 16)]
            for r in range(NS):
                hv = hv + hp[pl.ds(RPS * r + 16 * k, 16)]
            vn = (0.5 * vown[pl.ds(16 * k, 16)]
                  + 0.5 * v0own[pl.ds(16 * k, 16)]
                  + (0.1 * D) * hv + 0.001 * u)
            vown[pl.ds(16 * k, 16)] = vn
            return accv + vn

        lax.fori_loop(0, RPS // 16, nb, zf)
        pltpu.sync_copy(vown, v_sh.at[pl.ds(RPS * s, RPS)])
        plsc.subcore_barrier()

        pltpu.sync_copy(ep_sh, pbuf)
        acc2 = zf
        for r in range(NS):
            acc2 = acc2 + pbuf[pl.ds(16 * r, 16)]
        heb_tot = vsum(acc2)
        t_tot = (0.5 * t_run + 0.5 * u0 + (0.2 * D) * heb_tot
                 + 0.001 * N * u)
        u = 0.5 * u + 0.5 * u0 + (0.01 * D) * heb_tot + 0.001 * t_tot
        t_run = t_tot

    def cs(k, carry):
        x = cntloc[pl.ds(16 * k, 16)]
        inc = prefix_incl(x)
        rbfull[pl.ds(16 * k, 16)] = (carry + inc) - x
        return carry + splat(inc, 15)

    lax.fori_loop(0, N // 16, cs, jnp.zeros((16,), jnp.int32))

    def pb(a, c):
        rbchunk = rbfull[pl.ds(RPS * s + 16 * a, 16)]
        cchunk = cntloc[pl.ds(RPS * s + 16 * a, 16)]
        for b in range(16):
            off = 256 * a + 16 * b
            pos = jnp.where(iota < splat(cchunk, b),
                            splat(rbchunk, b) + iota,
                            jnp.broadcast_to(E, (16,)).astype(jnp.int32))
            j = 2 * a + (b // 8)
            pos2[j, pl.ds(16 * (b % 8), 16)] = pos
        return c

    lax.fori_loop(0, RPS // 16, pb, 0)

    def ob(j, c):
        pltpu.async_copy(ecur.at[pl.ds(128 * j, 128)],
                         out_sh.at[pos2.at[j]], sem)
        return c

    lax.fori_loop(0, NS, ob, 0)

    def ow(j, c):
        pltpu.make_async_copy(ecur.at[pl.ds(128 * j, 128)],
                              out_sh.at[pos2.at[j]], sem).wait()
        return c

    lax.fori_loop(0, NS, ow, 0)
    plsc.subcore_barrier()
    pltpu.sync_copy(out_sh.at[pl.ds((E // NS) * s, E // NS)],
                    out.at[pl.ds((E // NS) * s, E // NS)])


def _sc_recur(adjf, hicf, cnt, v0):
    mesh = plsc.VectorSubcoreMesh(core_axis_name="c", subcore_axis_name="s",
                                  num_cores=1)
    fn = pl.kernel(
        _sc_body,
        out_type=jax.ShapeDtypeStruct((E + OUTPAD,), jnp.float32),
        mesh=mesh,
        scratch_types=[
            pltpu.VMEM((SLOTS,), jnp.float32),
            pltpu.VMEM((SLOTS,), jnp.float32),
            pltpu.VMEM((SLOTS,), jnp.int32),
            pltpu.VMEM((NS, 128), jnp.int32),
            pltpu.VMEM((NS, 128), jnp.int32),
            pltpu.VMEM((NS, 128), jnp.int32),
            pltpu.VMEM((SLOTS,), jnp.float32),
            pltpu.VMEM((N,), jnp.int32),
            pltpu.VMEM((N,), jnp.int32),
            pltpu.VMEM((N,), jnp.float32),
            pltpu.VMEM((RPS,), jnp.float32),
            pltpu.VMEM((RPS,), jnp.float32),
            pltpu.VMEM((RPS,), jnp.float32),
            pltpu.VMEM((N,), jnp.float32),
            pltpu.VMEM((N,), jnp.float32),
            pltpu.VMEM((16,), jnp.float32),
            pltpu.VMEM((NS, 128), jnp.int32),
            pltpu.VMEM((NS * 16,), jnp.float32),
            pltpu.SemaphoreType.DMA,
            pltpu.VMEM_SHARED((N,), jnp.float32),
            pltpu.VMEM_SHARED((NS * N,), jnp.float32),
            pltpu.VMEM_SHARED((NS * 16,), jnp.float32),
            pltpu.VMEM_SHARED((E + OUTPAD,), jnp.float32),
        ],
    )
    return fn(adjf, hicf, cnt, v0)


def kernel(atoms, adjacency_map):
    hic, cnt, v0 = _tc_extract(atoms, adjacency_map)
    out = _sc_recur(adjacency_map.reshape(-1), hic.reshape(-1), cnt, v0)
    return out[:E].reshape(E, 1)
